# baseline (speedup 1.0000x reference)
import functools

import jax
import jax.numpy as jnp
from jax import lax
from jax.experimental import pallas as pl
from jax.experimental.pallas import tpu as pltpu
from jax.experimental.pallas import tpu_sc as plsc

D = 128
TEMP_INV = 10.0
EPS = 1e-8
NC = 2
NS = 16
NW = NC * NS
L = 16
CHUNK = 128


def _normalize_rows_bf16(z):
    n, d = z.shape
    bs = 8
    for cand in (4000, 2000, 1000, 800, 400, 200, 80, 40, 16, 8):
        if n % cand == 0:
            bs = cand
            break
    assert n % bs == 0

    def body(z_ref, o_ref):
        x = z_ref[...]
        ss = jnp.sum(x * x, axis=1, keepdims=True)
        inv = 1.0 / jnp.maximum(jnp.sqrt(ss), EPS)
        o_ref[...] = (x * inv).astype(jnp.bfloat16)

    return pl.pallas_call(
        body,
        grid=(n // bs,),
        in_specs=[pl.BlockSpec((bs, d), lambda i: (i, 0))],
        out_specs=pl.BlockSpec((bs, d), lambda i: (i, 0)),
        out_shape=jax.ShapeDtypeStruct((n, d), jnp.bfloat16),
    )(z)


def _make_sc_kernel(n_pairs, n_pos):
    assert n_pairs % (NW * CHUNK) == 0
    ppt = n_pairs // NW
    nch = ppt // CHUNK
    mesh = plsc.VectorSubcoreMesh(core_axis_name="c", subcore_axis_name="s")

    @functools.partial(
        pl.kernel,
        mesh=mesh,
        compiler_params=pltpu.CompilerParams(
            needs_layout_passes=False, use_tc_tiling_on_sc=False),
        out_type=[
            jax.ShapeDtypeStruct((NW, L), jnp.float32),
            jax.ShapeDtypeStruct((NW, L), jnp.float32),
        ],
        scratch_types=[
            pltpu.VMEM((ppt,), jnp.int32),
            pltpu.VMEM((ppt,), jnp.int32),
            pltpu.VMEM((CHUNK, D), jnp.bfloat16),
            pltpu.VMEM((CHUNK, D), jnp.bfloat16),
            pltpu.VMEM((L,), jnp.float32),
            pltpu.VMEM((L,), jnp.float32),
            pltpu.SemaphoreType.DMA,
            pltpu.SemaphoreType.DMA,
        ],
    )
    def sc_kernel(z_hbm, ii_hbm, jj_hbm, oexp_hbm, opos_hbm,
                  ii_v, jj_v, rows_i, rows_j,
                  accexp, accpos, sem_i, sem_j):
        wid = lax.axis_index("s") * NC + lax.axis_index("c")
        base = wid * ppt
        pltpu.sync_copy(ii_hbm.at[pl.ds(base, ppt)], ii_v)
        pltpu.sync_copy(jj_hbm.at[pl.ds(base, ppt)], jj_v)
        accexp[...] = jnp.zeros((L,), jnp.float32)
        accpos[...] = jnp.zeros((L,), jnp.float32)
        lane = lax.broadcasted_iota(jnp.int32, (L,), 0)

        def chunk_body(ch, _):
            off = ch * CHUNK
            cp_i = pltpu.async_copy(
                z_hbm.at[ii_v.at[pl.ds(off, CHUNK)]], rows_i, sem_i)
            cp_j = pltpu.async_copy(
                z_hbm.at[jj_v.at[pl.ds(off, CHUNK)]], rows_j, sem_j)
            cp_i.wait()
            cp_j.wait()

            def group_body(g, _):
                ab_vec = jnp.zeros((L,), jnp.float32)
                for k in range(L):
                    p = g * L + k
                    ab = jnp.zeros((L,), jnp.float32)
                    for s in range(D // (2 * L)):
                        a2 = rows_i[p, pl.ds(s * 2 * L, 2 * L)]
                        b2 = rows_j[p, pl.ds(s * 2 * L, 2 * L)]
                        a0, a1 = plsc.unpack(
                            a2, format=plsc.PackFormat.INTERLEAVED)
                        b0, b1 = plsc.unpack(
                            b2, format=plsc.PackFormat.INTERLEAVED)
                        ab = ab + a0 * b0
                        ab = ab + a1 * b1
                    ab_vec = jnp.where(lane == k, jnp.sum(ab), ab_vec)
                logit = ab_vec * TEMP_INV
                accexp[...] += jnp.exp(logit - 10.0)
                gidx = base + off + g * L + lane
                accpos[...] += jnp.where(gidx < n_pos, logit, 0.0)
                return 0

            lax.fori_loop(0, CHUNK // L, group_body, 0)
            return 0

        lax.fori_loop(0, nch, chunk_body, 0)
        pltpu.sync_copy(accexp, oexp_hbm.at[wid])
        pltpu.sync_copy(accpos, opos_hbm.at[wid])

    return sc_kernel


def kernel(z_all, pos_pairs, neg_pairs):
    n_pos = pos_pairs.shape[0]
    pairs = jnp.concatenate([pos_pairs, neg_pairs], axis=0)
    ii = pairs[:, 0]
    jj = pairs[:, 1]
    zn = _normalize_rows_bf16(z_all)
    sc = _make_sc_kernel(pairs.shape[0], n_pos)
    part_exp, part_pos = sc(zn, ii, jj)
    lse = 10.0 + jnp.log(jnp.sum(part_exp))
    return lse - jnp.sum(part_pos) / n_pos
